# Initial kernel scaffold; baseline (speedup 1.0000x reference)
import jax
import jax.numpy as jnp
from jax.experimental import pallas as pl


def kernel(xyz, features, Wq, bq, Wk, bk, Wv, bv, Wp1, bp1, p_g, p_b, p_m, p_v, Wp2, bp2, w_g1, w_b1, w_m1, w_v1, Ww1, bw1, w_g2, w_b2, w_m2, w_v2, Ww2, bw2):
    raise NotImplementedError("write your pallas kernel here")


# speedup vs baseline: 11.2144x; 11.2144x over previous
import functools

import jax
import jax.numpy as jnp
from jax import lax
from jax.experimental import pallas as pl
from jax.experimental.pallas import tpu as pltpu
from jax.experimental.pallas import tpu_sc as plsc

N = 10000
K = 16
C = 128
NP = 10112
EPS = 1e-5

RB = 128
GRID = NP // RB

NW = 32
PER_W = NP * K // NW
CH = 64
NCH = PER_W // CH

FAR = 1.0e6


def _prep_body(f_ref, x16_ref, wq_ref, wk_ref, wv_ref, bq_ref, bk_ref, bv_ref,
               wp1_ref, bp1_ref, psc_ref, psh_ref, wp2_ref, bp2_ref,
               q_ref, kv_ref):
    f = f_ref[...]
    q = jnp.dot(f, wq_ref[...], preferred_element_type=jnp.float32) + bq_ref[...]
    k = jnp.dot(f, wk_ref[...], preferred_element_type=jnp.float32) + bk_ref[...]
    v = jnp.dot(f, wv_ref[...], preferred_element_type=jnp.float32) + bv_ref[...]
    x16 = x16_ref[...]
    p1 = jnp.dot(x16, wp1_ref[...], preferred_element_type=jnp.float32) + bp1_ref[...]
    p1 = jnp.maximum(p1 * psc_ref[...] + psh_ref[...], 0.0)
    p2 = jnp.dot(p1, wp2_ref[...], preferred_element_type=jnp.float32) + bp2_ref[...]
    q_ref[...] = q
    kv_ref[...] = jnp.concatenate([k + p2, v + p2], axis=1)


def _knn_body(xr_ref, xa_ref, idx_ref):
    xr = xr_ref[...]
    xa = xa_ref[...]
    sq_r = jnp.sum(xr * xr, axis=1, keepdims=True)
    sq_a = jnp.sum(xa * xa, axis=0, keepdims=True)
    d2 = sq_r + sq_a - 2.0 * jnp.dot(xr, xa, preferred_element_type=jnp.float32)
    cols = lax.broadcasted_iota(jnp.int32, (RB, NP), 1)
    lane16 = lax.broadcasted_iota(jnp.int32, (RB, K), 1)

    def body(t, carry):
        d2, acc = carry
        m = jnp.min(d2, axis=1, keepdims=True)
        am = jnp.min(jnp.where(d2 <= m, cols, NP), axis=1, keepdims=True)
        acc = jnp.where(lane16 == t, am, acc)
        d2 = jnp.where(cols == am, jnp.inf, d2)
        return d2, acc

    _, acc = lax.fori_loop(0, K, body, (d2, jnp.zeros((RB, K), jnp.int32)))
    idx_ref[...] = acc


def _attn_body(kvg_ref, q_ref, w1sc_ref, w1sh_ref, ww1_ref, bw1_ref,
               w2sc_ref, w2sh_ref, ww2_ref, bw2_ref, t16_ref, out_ref):
    kv = kvg_ref[...]
    kp = kv[:, :C]
    vp = kv[:, C:]
    q = q_ref[...]

    qb = jnp.broadcast_to(q[:, None, :], (RB, K, C)).reshape(RB * K, C)
    w = kp - qb
    w = jnp.maximum(w * w1sc_ref[...] + w1sh_ref[...], 0.0)
    w = jnp.dot(w, ww1_ref[...], preferred_element_type=jnp.float32) + bw1_ref[...]
    w = jnp.maximum(w * w2sc_ref[...] + w2sh_ref[...], 0.0)
    w = jnp.dot(w, ww2_ref[...], preferred_element_type=jnp.float32) + bw2_ref[...]

    w3 = w.reshape(RB, K, 16)
    m = jnp.max(w3, axis=1, keepdims=True)
    e = jnp.exp(w3 - m)
    sm = (e / jnp.sum(e, axis=1, keepdims=True)).reshape(RB * K, 16)
    wfull = jnp.dot(sm, t16_ref[...], preferred_element_type=jnp.float32)

    out_ref[...] = jnp.sum((vp * wfull).reshape(RB, K, C), axis=1)


def _sc_gather(kv_t, idxw):
    mesh = plsc.VectorSubcoreMesh(core_axis_name="c", subcore_axis_name="s")

    @functools.partial(
        pl.kernel, mesh=mesh,
        out_type=jax.ShapeDtypeStruct((NP * K, 2 * C), jnp.float32),
        scratch_types=[pltpu.VMEM((NCH, CH), jnp.int32),
                       pltpu.VMEM((CH, 2 * C), jnp.float32),
                       pltpu.SemaphoreType.DMA],
    )
    def gather_kernel(kv_hbm, idx_hbm, kvg_out, idx_v, rows_kv, sem):
        wid = lax.axis_index("s") * 2 + lax.axis_index("c")
        pltpu.sync_copy(idx_hbm.at[wid], idx_v)
        base = wid * PER_W

        def body(c, carry):
            pltpu.async_copy(kv_hbm.at[idx_v.at[c]], rows_kv, sem).wait()
            pltpu.sync_copy(rows_kv, kvg_out.at[pl.ds(base + c * CH, CH)])
            return carry

        lax.fori_loop(0, NCH, body, 0)

    return gather_kernel(kv_t, idxw)


def kernel(xyz, features, Wq, bq, Wk, bk, Wv, bv, Wp1, bp1, p_g, p_b, p_m, p_v,
           Wp2, bp2, w_g1, w_b1, w_m1, w_v1, Ww1, bw1, w_g2, w_b2, w_m2, w_v2,
           Ww2, bw2):
    f32 = jnp.float32

    xyz0 = xyz[0]
    xyz8 = jnp.zeros((NP, 8), f32)
    xyz8 = xyz8.at[:N, :3].set(xyz0)
    xyz8 = xyz8.at[N:, 0].set(FAR)
    xyz8t = xyz8.T
    x16 = jnp.concatenate([xyz8, jnp.zeros((NP, 8), f32)], axis=1)

    featT = jnp.pad(features[0].T, ((0, NP - N), (0, 0)))

    def bn_fold(g, b, m, v):
        sc = g / jnp.sqrt(v + EPS)
        return sc, b - m * sc

    psc, psh = bn_fold(p_g, p_b, p_m, p_v)
    w1sc, w1sh = bn_fold(w_g1, w_b1, w_m1, w_v1)
    w2sc, w2sh = bn_fold(w_g2, w_b2, w_m2, w_v2)

    wp1p = jnp.zeros((16, 16), f32).at[:3, :3].set(Wp1.T)
    bp1p = jnp.zeros((1, 16), f32).at[0, :3].set(bp1)
    pscp = jnp.ones((1, 16), f32).at[0, :3].set(psc)
    pshp = jnp.zeros((1, 16), f32).at[0, :3].set(psh)
    wp2p = jnp.zeros((16, C), f32).at[:3, :].set(Wp2.T)

    t16 = (lax.broadcasted_iota(jnp.int32, (16, C), 1) % 16 ==
           lax.broadcasted_iota(jnp.int32, (16, C), 0)).astype(f32)

    q_t, kv_t = pl.pallas_call(
        _prep_body,
        grid=(GRID,),
        in_specs=[pl.BlockSpec((RB, C), lambda i: (i, 0)),
                  pl.BlockSpec((RB, 16), lambda i: (i, 0))] +
                 [pl.BlockSpec((C, C), lambda i: (0, 0))] * 3 +
                 [pl.BlockSpec((1, C), lambda i: (0, 0))] * 3 +
                 [pl.BlockSpec((16, 16), lambda i: (0, 0)),
                  pl.BlockSpec((1, 16), lambda i: (0, 0)),
                  pl.BlockSpec((1, 16), lambda i: (0, 0)),
                  pl.BlockSpec((1, 16), lambda i: (0, 0)),
                  pl.BlockSpec((16, C), lambda i: (0, 0)),
                  pl.BlockSpec((1, C), lambda i: (0, 0))],
        out_specs=[pl.BlockSpec((RB, C), lambda i: (i, 0)),
                   pl.BlockSpec((RB, 2 * C), lambda i: (i, 0))],
        out_shape=[jax.ShapeDtypeStruct((NP, C), f32),
                   jax.ShapeDtypeStruct((NP, 2 * C), f32)],
    )(featT, x16, Wq.T, Wk.T, Wv.T, bq[None], bk[None], bv[None],
      wp1p, bp1p, pscp, pshp, wp2p, bp2[None])

    idx = pl.pallas_call(
        _knn_body,
        grid=(GRID,),
        in_specs=[pl.BlockSpec((RB, 8), lambda i: (i, 0)),
                  pl.BlockSpec((8, NP), lambda i: (0, 0))],
        out_specs=pl.BlockSpec((RB, K), lambda i: (i, 0)),
        out_shape=jax.ShapeDtypeStruct((NP, K), jnp.int32),
    )(xyz8, xyz8t)

    idxw = idx.reshape(NW, NCH, CH)
    kvg = _sc_gather(kv_t, idxw)

    wspec = lambda shape: pl.BlockSpec(shape, lambda i: (0, 0))
    out = pl.pallas_call(
        _attn_body,
        grid=(GRID,),
        in_specs=[pl.BlockSpec((RB * K, 2 * C), lambda i: (i, 0)),
                  pl.BlockSpec((RB, C), lambda i: (i, 0)),
                  wspec((1, C)), wspec((1, C)),
                  wspec((C, 16)), wspec((1, 16)), wspec((1, 16)), wspec((1, 16)),
                  wspec((16, 16)), wspec((1, 16)), wspec((16, C))],
        out_specs=pl.BlockSpec((RB, C), lambda i: (i, 0)),
        out_shape=jax.ShapeDtypeStruct((NP, C), f32),
    )(kvg, q_t,
      w1sc[None], w1sh[None],
      Ww1.T, bw1[None], w2sc[None], w2sh[None],
      Ww2.T, bw2[None], t16)

    return out[:N].T[None]
